# baseline (device time: 20139 ns/iter reference)
import jax
import jax.numpy as jnp
from jax import lax
from jax.experimental import pallas as pl
from jax.experimental.pallas import tpu as pltpu

X_SIZE = 2
K = 4


def kernel(x):
    m, n = x.shape
    half = n // X_SIZE
    rows = m // K

    def body(x_hbm, out_hbm, xv, sbuf, rbuf, rv,
             load_sems, store_sems, loc_sem, send_sems, recv_sems):
        my_x = lax.axis_index("x")
        my_y = lax.axis_index("y")
        my_z = lax.axis_index("z")
        ox = 1 - my_x
        partner = (ox, my_y, my_z)

        barrier_sem = pltpu.get_barrier_semaphore()
        pl.semaphore_signal(
            barrier_sem, inc=1,
            device_id=partner, device_id_type=pl.DeviceIdType.MESH,
        )
        pl.semaphore_wait(barrier_sem, 1)

        loc = pltpu.make_async_copy(
            x_hbm.at[:, pl.ds(my_x * half, half)],
            out_hbm.at[pl.ds(my_x * m, m), :],
            loc_sem,
        )
        loc.start()

        loads = []
        for h in range(K):
            sl = pl.ds(h * rows, rows)
            ld = pltpu.make_async_copy(
                x_hbm.at[sl, pl.ds(ox * half, half)],
                xv.at[sl, :],
                load_sems.at[h],
            )
            ld.start()
            loads.append(ld)

        rdmas = []
        for h in range(K):
            sl = pl.ds(h * rows, rows)
            loads[h].wait()
            sbuf[sl, :] = xv[sl, :].astype(jnp.bfloat16)
            rdma = pltpu.make_async_remote_copy(
                src_ref=sbuf.at[sl, :],
                dst_ref=rbuf.at[sl, :],
                send_sem=send_sems.at[h],
                recv_sem=recv_sems.at[h],
                device_id=partner,
                device_id_type=pl.DeviceIdType.MESH,
            )
            rdma.start()
            rdmas.append(rdma)

        stores = []
        for h, rdma in enumerate(rdmas):
            sl = pl.ds(h * rows, rows)
            rdma.wait()
            rv[sl, :] = rbuf[sl, :].astype(jnp.float32)
            st = pltpu.make_async_copy(
                rv.at[sl, :],
                out_hbm.at[pl.ds(ox * m + h * rows, rows), :],
                store_sems.at[h],
            )
            st.start()
            stores.append(st)
        for st in stores:
            st.wait()
        loc.wait()

    return pl.pallas_call(
        body,
        out_shape=jax.ShapeDtypeStruct((X_SIZE * m, half), x.dtype),
        in_specs=[pl.BlockSpec(memory_space=pltpu.MemorySpace.HBM)],
        out_specs=pl.BlockSpec(memory_space=pltpu.MemorySpace.HBM),
        scratch_shapes=[
            pltpu.VMEM((m, half), jnp.float32),
            pltpu.VMEM((m, half), jnp.bfloat16),
            pltpu.VMEM((m, half), jnp.bfloat16),
            pltpu.VMEM((m, half), jnp.float32),
            pltpu.SemaphoreType.DMA((K,)),
            pltpu.SemaphoreType.DMA((K,)),
            pltpu.SemaphoreType.DMA,
            pltpu.SemaphoreType.DMA((K,)),
            pltpu.SemaphoreType.DMA((K,)),
        ],
        compiler_params=pltpu.CompilerParams(collective_id=0),
    )(x)


# device time: 19393 ns/iter; 1.0385x vs baseline; 1.0385x over previous
import jax
import jax.numpy as jnp
from jax import lax
from jax.experimental import pallas as pl
from jax.experimental.pallas import tpu as pltpu

X_SIZE = 2
K = 4


def kernel(x):
    m, n = x.shape
    half = n // X_SIZE
    rows = m // K

    def body(x_ref, out_ref, sbuf, rbuf, send_sems, recv_sems):
        my_x = lax.axis_index("x")
        my_y = lax.axis_index("y")
        my_z = lax.axis_index("z")
        ox = 1 - my_x
        partner = (ox, my_y, my_z)

        barrier_sem = pltpu.get_barrier_semaphore()
        pl.semaphore_signal(
            barrier_sem, inc=1,
            device_id=partner, device_id_type=pl.DeviceIdType.MESH,
        )
        pl.semaphore_wait(barrier_sem, 1)

        rdmas = []
        for h in range(K):
            sl = pl.ds(h * rows, rows)
            sbuf[sl, :] = x_ref[sl, pl.ds(ox * half, half)].astype(jnp.bfloat16)
            rdma = pltpu.make_async_remote_copy(
                src_ref=sbuf.at[sl, :],
                dst_ref=rbuf.at[sl, :],
                send_sem=send_sems.at[h],
                recv_sem=recv_sems.at[h],
                device_id=partner,
                device_id_type=pl.DeviceIdType.MESH,
            )
            rdma.start()
            rdmas.append(rdma)

        out_ref[pl.ds(my_x * m, m), :] = x_ref[:, pl.ds(my_x * half, half)]

        for h, rdma in enumerate(rdmas):
            rdma.wait()
            sl = pl.ds(h * rows, rows)
            out_ref[pl.ds(ox * m + h * rows, rows), :] = rbuf[sl, :].astype(
                jnp.float32
            )

    return pl.pallas_call(
        body,
        out_shape=jax.ShapeDtypeStruct((X_SIZE * m, half), x.dtype),
        in_specs=[pl.BlockSpec(memory_space=pltpu.VMEM)],
        out_specs=pl.BlockSpec(memory_space=pltpu.VMEM),
        scratch_shapes=[
            pltpu.VMEM((m, half), jnp.bfloat16),
            pltpu.VMEM((m, half), jnp.bfloat16),
            pltpu.SemaphoreType.DMA((K,)),
            pltpu.SemaphoreType.DMA((K,)),
        ],
        compiler_params=pltpu.CompilerParams(collective_id=0),
    )(x)
